# Initial kernel scaffold; baseline (speedup 1.0000x reference)
import jax
import jax.numpy as jnp
from jax.experimental import pallas as pl


def kernel(x, edge_index, params):
    raise NotImplementedError("write your pallas kernel here")


# speedup vs baseline: 9.9526x; 9.9526x over previous
import functools

import jax
import jax.numpy as jnp
from jax import lax
from jax.experimental import pallas as pl
from jax.experimental.pallas import tpu as pltpu
from jax.experimental.pallas import tpu_sc as plsc

N = 10000
E = 320000
FIN = 128
NH = 38
NC = 64
KCHEB = 6
KMIX = 2
KSG = 5

NPAD = 10240
RPT = NPAD // 16
HS = 2
PW = 2 * HS * 16
CSZ = 128
CH = 157
EPAD = 16 * CH * CSZ

_MESH = dict(core_axis_name="c", subcore_axis_name="s")

_DNUMS = lax.GatherDimensionNumbers(
    offset_dims=(), collapsed_slice_dims=(0,), start_index_map=(0,))


def _bcast(vec, i):
    idx = jnp.full((16, 1), i, jnp.int32)
    return lax.gather(vec, idx, _DNUMS, (1,),
                      mode=lax.GatherScatterMode.PROMISE_IN_BOUNDS)


def _hist_body(srcT, dstT, deg_out, cnt_out, hist, eidx, ones, zbuf, sbuf):
    cid = lax.axis_index("c")
    sid = lax.axis_index("s")
    r0 = sid * RPT

    @pl.when(cid == 0)
    def _():
        pltpu.sync_copy(srcT.at[sid], eidx)

    @pl.when(cid == 1)
    def _():
        pltpu.sync_copy(dstT.at[sid], eidx)

    def zloop(j, _):
        zbuf[pl.ds(j * 16, 16)] = jnp.zeros((16,), jnp.float32)
        ones[pl.ds(j * 16, 16)] = jnp.ones((16,), jnp.float32)
        return _
    lax.fori_loop(0, CSZ // 16, zloop, 0)

    def z2loop(j, _):
        pltpu.sync_copy(zbuf, hist.at[pl.ds(r0 + j * CSZ, CSZ)])
        return _
    lax.fori_loop(0, RPT // CSZ, z2loop, 0)
    plsc.subcore_barrier()

    def chunk(c, _):
        pltpu.sync_copy(ones, hist.at[eidx.at[c]], add=True)
        return _
    lax.fori_loop(0, CH, chunk, 0)
    plsc.subcore_barrier()

    pltpu.sync_copy(hist.at[pl.ds(r0, RPT)], sbuf)

    @pl.when(cid == 0)
    def _():
        pltpu.sync_copy(sbuf, deg_out.at[pl.ds(r0, RPT)])

    @pl.when(cid == 1)
    def _():
        pltpu.sync_copy(sbuf, cnt_out.at[pl.ds(r0, RPT)])


def _histograms(srcT, dstT):
    f = pl.kernel(
        _hist_body,
        out_type=(jax.ShapeDtypeStruct((NPAD,), jnp.float32),
                  jax.ShapeDtypeStruct((NPAD,), jnp.float32)),
        mesh=plsc.VectorSubcoreMesh(**_MESH),
        scratch_types=dict(
            hist=pltpu.VMEM_SHARED((NPAD,), jnp.float32),
            eidx=pltpu.VMEM((CH, CSZ), jnp.int32),
            ones=pltpu.VMEM((CSZ,), jnp.float32),
            zbuf=pltpu.VMEM((CSZ,), jnp.float32),
            sbuf=pltpu.VMEM((RPT,), jnp.float32),
        ),
    )
    return f(srcT, dstT)


def _wts_body(srcT, dstT, disc, dissg, invc,
              wch_out, wsa_out, wsg_out,
              esrc, edst, wb1, wb2, tb1, tb2):
    cid = lax.axis_index("c")
    sid = lax.axis_index("s")

    pltpu.sync_copy(srcT.at[sid], esrc)
    pltpu.sync_copy(dstT.at[sid], edst)

    def lg(tb, v):
        return plsc.load_gather(tb, [lax.shift_right_logical(v, 7),
                                     lax.bitwise_and(v, 127)])

    @pl.when(cid == 0)
    def _():
        pltpu.sync_copy(disc, tb1)
        pltpu.sync_copy(invc, tb2)

        def loop(c, _):
            def g(gi, _2):
                s = esrc[c, pl.ds(gi * 16, 16)]
                d = edst[c, pl.ds(gi * 16, 16)]
                vs = lg(tb1, s)
                vd = lg(tb1, d)
                wb1[c, pl.ds(gi * 16, 16)] = -(vs * vd)
                wb2[c, pl.ds(gi * 16, 16)] = lg(tb2, d)
                return _2
            lax.fori_loop(0, CSZ // 16, g, 0)
            return _
        lax.fori_loop(0, CH, loop, 0)
        pltpu.sync_copy(wb1, wch_out.at[sid])
        pltpu.sync_copy(wb2, wsa_out.at[sid])

    @pl.when(cid == 1)
    def _():
        pltpu.sync_copy(dissg, tb1)

        def loop(c, _):
            def g(gi, _2):
                s = esrc[c, pl.ds(gi * 16, 16)]
                d = edst[c, pl.ds(gi * 16, 16)]
                vs = lg(tb1, s)
                vd = lg(tb1, d)
                wb1[c, pl.ds(gi * 16, 16)] = vs * vd
                return _2
            lax.fori_loop(0, CSZ // 16, g, 0)
            return _
        lax.fori_loop(0, CH, loop, 0)
        pltpu.sync_copy(wb1, wsg_out.at[sid])


def _edge_weights(srcT, dstT, disc, dissg, invc):
    f = pl.kernel(
        _wts_body,
        compiler_params=pltpu.CompilerParams(needs_layout_passes=False),
        out_type=(jax.ShapeDtypeStruct((16, CH, CSZ), jnp.float32),
                  jax.ShapeDtypeStruct((16, CH, CSZ), jnp.float32),
                  jax.ShapeDtypeStruct((16, CH, CSZ), jnp.float32)),
        mesh=plsc.VectorSubcoreMesh(**_MESH),
        scratch_types=dict(
            esrc=pltpu.VMEM((CH, CSZ), jnp.int32),
            edst=pltpu.VMEM((CH, CSZ), jnp.int32),
            wb1=pltpu.VMEM((CH, CSZ), jnp.float32),
            wb2=pltpu.VMEM((CH, CSZ), jnp.float32),
            tb1=pltpu.VMEM((NPAD // 128, 128), jnp.float32),
            tb2=pltpu.VMEM((NPAD // 128, 128), jnp.float32),
        ),
    )
    return f(srcT, dstT, disc.reshape(NPAD // 128, 128),
             dissg.reshape(NPAD // 128, 128),
             invc.reshape(NPAD // 128, 128))


def _make_prop_body(ncheb):
    sg_i = ncheb
    sa_i = ncheb + 1

    def body(srcT, dstT, wch, wsa, wsg, ycat, d2t,
             cheb_out, sage_out, sg_out,
             t0, t1,
             esrc, edst, eww, rows, ybuf, d2v, sem):
        cid = lax.axis_index("c")
        sid = lax.axis_index("s")
        r0 = sid * RPT
        tabs = [t0, t1]

        pltpu.sync_copy(srcT.at[sid], esrc)
        pltpu.sync_copy(dstT.at[sid], edst)
        pltpu.sync_copy(d2t.at[pl.ds(r0, RPT)], d2v)

        def load_y(k, buf):
            pltpu.sync_copy(
                ycat.at[pl.ds(r0, RPT), pl.ds(k * 2 * HS + cid * HS, HS)],
                buf)

        def store_t(buf, t):
            pltpu.sync_copy(buf, t.at[pl.ds(r0, RPT)])

        def load_t(t, buf):
            pltpu.sync_copy(t.at[pl.ds(r0, RPT)], buf)

        def writeout(t, out):
            pltpu.sync_copy(
                t.at[pl.ds(r0, RPT)],
                out.at[pl.ds(r0, RPT), pl.ds(cid * HS, HS)])

        def init_copy_y(k, t):
            load_y(k, ybuf)
            store_t(ybuf, t)

        def init_zero(t):
            def l(r, _):
                for kk in range(HS):
                    ybuf[r, kk, :] = jnp.zeros((16,), jnp.float32)
                return _
            lax.fori_loop(0, RPT, l, 0)
            store_t(ybuf, t)

        def init_d2(tsrc, t):
            load_t(tsrc, ybuf)

            def l(g, _):
                dv = d2v[pl.ds(g * 16, 16)]
                for i in range(16):
                    b = _bcast(dv, i)
                    r = g * 16 + i
                    for kk in range(HS):
                        ybuf[r, kk, :] = ybuf[r, kk, :] * b
                return _
            lax.fori_loop(0, RPT // 16, l, 0)
            store_t(ybuf, t)

        def set_w(w_hbm):
            pltpu.sync_copy(w_hbm.at[sid], eww)

        def prop(tsrc, tdst, scale):
            def chunk(c, _):
                pltpu.async_copy(tsrc.at[esrc.at[c]], rows, sem).wait()

                def sc(g, _2):
                    wv = eww[c, pl.ds(g * 16, 16)]
                    if scale != 1.0:
                        wv = wv * scale
                    for i in range(16):
                        b = _bcast(wv, i)
                        r = g * 16 + i
                        for kk in range(HS):
                            rows[r, kk, :] = rows[r, kk, :] * b
                    return _2
                lax.fori_loop(0, CSZ // 16, sc, 0)
                pltpu.sync_copy(rows, tdst.at[edst.at[c]], add=True)
                return _
            lax.fori_loop(0, CH, chunk, 0)

        bar = plsc.subcore_barrier

        def init_y_minus_rmw(k, t):
            load_y(k, ybuf)
            for sub in range(RPT // CSZ):
                pltpu.sync_copy(t.at[pl.ds(r0 + sub * CSZ, CSZ)], rows)

                def l(r, _, sub=sub):
                    for kk in range(HS):
                        rr = sub * CSZ + r
                        ybuf[rr, kk, :] = ybuf[rr, kk, :] - rows[r, kk, :]
                    return _
                lax.fori_loop(0, CSZ, l, 0)
            store_t(ybuf, t)

        set_w(wch)
        init_copy_y(ncheb - 1, tabs[0])
        bar()
        cur = 0
        first = True
        for k in range(ncheb - 2, -1, -1):
            t = 1 - cur
            if first:
                init_copy_y(k, tabs[t])
                first = False
            else:
                init_y_minus_rmw(k, tabs[t])
            bar()
            prop(tabs[cur], tabs[t], 2.0 if k > 0 else 1.0)
            bar()
            cur = t
        cheb_t = cur

        writeout(tabs[cheb_t], cheb_out)
        init_copy_y(sa_i, tabs[1 - cheb_t])
        init_zero(tabs[cheb_t])
        set_w(wsa)
        bar()
        prop(tabs[1 - cheb_t], tabs[cheb_t], 1.0)
        bar()

        writeout(tabs[cheb_t], sage_out)
        init_copy_y(sg_i, tabs[1 - cheb_t])
        set_w(wsg)
        bar()
        h = 1 - cheb_t
        for _ in range(KSG):
            o = 1 - h
            init_d2(tabs[h], tabs[o])
            bar()
            prop(tabs[h], tabs[o], 1.0)
            bar()
            h = o
        writeout(tabs[h], sg_out)

    return body


def _make_prop_kernel(ncheb, ntab):
    f = pl.kernel(
        _make_prop_body(ncheb),
        compiler_params=pltpu.CompilerParams(use_tc_tiling_on_sc=False),
        out_type=(jax.ShapeDtypeStruct((NPAD, 2 * HS, 16), jnp.float32),
                  jax.ShapeDtypeStruct((NPAD, 2 * HS, 16), jnp.float32),
                  jax.ShapeDtypeStruct((NPAD, 2 * HS, 16), jnp.float32)),
        mesh=plsc.VectorSubcoreMesh(**_MESH),
        scratch_types=dict(
            t0=pltpu.VMEM_SHARED((NPAD, HS, 16), jnp.float32),
            t1=pltpu.VMEM_SHARED((NPAD, HS, 16), jnp.float32),
            esrc=pltpu.VMEM((CH, CSZ), jnp.int32),
            edst=pltpu.VMEM((CH, CSZ), jnp.int32),
            eww=pltpu.VMEM((CH, CSZ), jnp.float32),
            rows=pltpu.VMEM((CSZ, HS, 16), jnp.float32),
            ybuf=pltpu.VMEM((RPT, HS, 16), jnp.float32),
            d2v=pltpu.VMEM((RPT,), jnp.float32),
            sem=pltpu.SemaphoreType.DMA,
        ),
    )
    return f


def _prep_body(deg_ref, cnt_ref, disc, dissg, d2o, invc):
    rows = lax.broadcasted_iota(jnp.int32, (NPAD // 128, 128), 0)
    cols = lax.broadcasted_iota(jnp.int32, (NPAD // 128, 128), 1)
    mask = (rows * 128 + cols) < N
    dg = deg_ref[...]
    cn = cnt_ref[...]
    disc[...] = jnp.where(
        jnp.logical_and(dg > 0, mask),
        lax.rsqrt(jnp.where(dg > 0, dg, 1.0)), 0.0)
    dsg = jnp.where(mask, lax.rsqrt(cn + 1.0), 0.0)
    dissg[...] = dsg
    d2o[...] = dsg * dsg
    invc[...] = jnp.where(mask, 1.0 / jnp.maximum(cn, 1.0), 0.0)


def _prep(deg, cnt):
    shp = jax.ShapeDtypeStruct((NPAD // 128, 128), jnp.float32)
    f = pl.pallas_call(
        _prep_body,
        out_shape=(shp, shp, shp, shp),
    )
    o = f(deg.reshape(NPAD // 128, 128), cnt.reshape(NPAD // 128, 128))
    return tuple(x.reshape(NPAD) for x in o)


def _mm_body(x_ref, w_ref, o_ref):
    o_ref[...] = jnp.dot(x_ref[...], w_ref[...],
                         preferred_element_type=jnp.float32)


def _mm(x, w):
    K, M = w.shape
    return pl.pallas_call(
        _mm_body,
        grid=(16,),
        in_specs=[pl.BlockSpec((RPT, K), lambda i: (i, 0)),
                  pl.BlockSpec((K, M), lambda i: (0, 0))],
        out_specs=pl.BlockSpec((RPT, M), lambda i: (i, 0)),
        out_shape=jax.ShapeDtypeStruct((NPAD, M), jnp.float32),
    )(x, w)


def _elu(v):
    return jnp.where(v > 0, v, jnp.exp(jnp.minimum(v, 0.0)) - 1.0)


def _combine(ch, sa, sg, rr, bias, final):
    a = ch[...] + bias[0:1, :]
    if not final:
        a = _elu(a)
    b = _elu(sa[...] + bias[1:2, :] + rr[...])
    c = _elu(sg[...] + bias[2:3, :])
    return (a + b + c) * jnp.float32(1.0 / 3.0)


def _cmb_mm_body(ch, sa, sg, rr, bias, w_ref, o_ref):
    xn = _combine(ch, sa, sg, rr, bias, False)
    o_ref[...] = jnp.dot(xn, w_ref[...], preferred_element_type=jnp.float32)


def _cmb_mm(ch, sa, sg, rr, bias, w):
    K, M = w.shape
    t = pl.BlockSpec((RPT, K), lambda i: (i, 0))
    return pl.pallas_call(
        _cmb_mm_body,
        grid=(16,),
        in_specs=[t, t, t, t,
                  pl.BlockSpec((3, K), lambda i: (0, 0)),
                  pl.BlockSpec((K, M), lambda i: (0, 0))],
        out_specs=pl.BlockSpec((RPT, M), lambda i: (i, 0)),
        out_shape=jax.ShapeDtypeStruct((NPAD, M), jnp.float32),
    )(ch, sa, sg, rr, bias, w)


def _mix_mm_body(ch, sa, sg, rr, bias, x0_ref, wt_ref, wb_ref, o_ref):
    xn = _combine(ch, sa, sg, rr, bias, False)
    o_ref[...] = (jnp.dot(xn, wt_ref[...], preferred_element_type=jnp.float32)
                  + jnp.dot(x0_ref[...], wb_ref[...],
                            preferred_element_type=jnp.float32))


def _mix_mm(ch, sa, sg, rr, bias, x0, wt, wb):
    K, M = wt.shape
    K0 = wb.shape[0]
    t = pl.BlockSpec((RPT, K), lambda i: (i, 0))
    return pl.pallas_call(
        _mix_mm_body,
        grid=(16,),
        in_specs=[t, t, t, t,
                  pl.BlockSpec((3, K), lambda i: (0, 0)),
                  pl.BlockSpec((RPT, K0), lambda i: (i, 0)),
                  pl.BlockSpec((K, M), lambda i: (0, 0)),
                  pl.BlockSpec((K0, M), lambda i: (0, 0))],
        out_specs=pl.BlockSpec((RPT, M), lambda i: (i, 0)),
        out_shape=jax.ShapeDtypeStruct((NPAD, M), jnp.float32),
    )(ch, sa, sg, rr, bias, x0, wt, wb)


def _fin_body(ch, sa, sg, rr, bias, o_ref):
    o_ref[...] = _combine(ch, sa, sg, rr, bias, True)


def _final(ch, sa, sg, rr, bias):
    K = NC
    t = pl.BlockSpec((RPT, K), lambda i: (i, 0))
    return pl.pallas_call(
        _fin_body,
        grid=(16,),
        in_specs=[t, t, t, t, pl.BlockSpec((3, K), lambda i: (0, 0))],
        out_specs=pl.BlockSpec((RPT, K), lambda i: (i, 0)),
        out_shape=jax.ShapeDtypeStruct((NPAD, K), jnp.float32),
    )(ch, sa, sg, rr, bias)


def _padw(w, rows, cols):
    return jnp.pad(w, ((0, rows - w.shape[0]), (0, cols - w.shape[1])))


def _pack_layer(lyr, fin_pad):
    ws = ([_padw(w, fin_pad, PW) for w in lyr['cheb_W']]
          + [_padw(lyr['sg_W'], fin_pad, PW),
             _padw(lyr['sage_Wl'], fin_pad, PW),
             _padw(lyr['sage_Wr'], fin_pad, PW)])
    wcat = jnp.concatenate(ws, axis=1)
    bias = jnp.stack([
        jnp.pad(lyr['cheb_b'], (0, PW - lyr['cheb_b'].shape[0])),
        jnp.pad(lyr['sage_bl'], (0, PW - lyr['sage_bl'].shape[0])),
        jnp.pad(lyr['sg_b'], (0, PW - lyr['sg_b'].shape[0]))])
    return wcat, bias


def kernel(x, edge_index, params):
    src = edge_index[0]
    dst = edge_index[1]
    npad_e = EPAD - E
    fill = (N + (jnp.arange(npad_e, dtype=jnp.int32) % 8)).astype(src.dtype)
    srcT = jnp.concatenate([src, fill]).reshape(16, CH, CSZ)
    dstT = jnp.concatenate([dst, fill]).reshape(16, CH, CSZ)

    x_pad = jnp.pad(x, ((0, NPAD - N), (0, 0)))

    deg, cnt = _histograms(srcT, dstT)
    disc, dissg, d2t, invc = _prep(deg, cnt)
    wch, wsa, wsg = _edge_weights(srcT, dstT, disc, dissg, invc)

    prop6 = _make_prop_kernel(KCHEB, 8)
    prop2 = _make_prop_kernel(KMIX, 4)

    nt6 = KCHEB + 3
    wcat0, bias0 = _pack_layer(params['layers'][0], FIN)
    y = _mm(x_pad, wcat0)

    def run_layer(y):
        ysc = y[:, :(nt6 - 1) * PW].reshape(NPAD, (nt6 - 1) * 2 * HS, 16)
        rr = y[:, (nt6 - 1) * PW:]
        ch, sa, sg = prop6(srcT, dstT, wch, wsa, wsg, ysc, d2t)
        return (ch.reshape(NPAD, PW), sa.reshape(NPAD, PW),
                sg.reshape(NPAD, PW), rr)

    for li in (1, 2):
        ch, sa, sg, rr = run_layer(y)
        wcat, bias = _pack_layer(params['layers'][li], PW)
        bias_prev = (bias0 if li == 1 else bias1)
        y = _cmb_mm(ch, sa, sg, rr, bias_prev, wcat)
        if li == 1:
            bias1 = bias
        else:
            bias2 = bias

    ch, sa, sg, rr = run_layer(y)
    m = params['mix']
    wmt = [_padw(w[:NH], PW, NC) for w in m['cheb_W']] + [
        _padw(m['sg_W'][:NH], PW, NC),
        _padw(m['sage_Wl'][:NH], PW, NC),
        _padw(m['sage_Wr'][:NH], PW, NC)]
    wmb = [w[NH:] for w in m['cheb_W']] + [
        m['sg_W'][NH:], m['sage_Wl'][NH:], m['sage_Wr'][NH:]]
    wt = jnp.concatenate(wmt, axis=1)
    wb = jnp.concatenate(wmb, axis=1)
    bias_m = jnp.stack([m['cheb_b'], m['sage_bl'], m['sg_b']])
    ym = _mix_mm(ch, sa, sg, rr, bias2, x_pad, wt, wb)

    nt2 = KMIX + 3
    ysc = ym[:, :(nt2 - 1) * NC].reshape(NPAD, (nt2 - 1) * 2 * HS, 16)
    rrm = ym[:, (nt2 - 1) * NC:]
    chm, sam, sgm = prop2(srcT, dstT, wch, wsa, wsg, ysc, d2t)
    out = _final(chm.reshape(NPAD, NC), sam.reshape(NPAD, NC),
                 sgm.reshape(NPAD, NC), rrm, bias_m)
    return out[:N]
